# baseline (device time: 110328 ns/iter reference)
import jax
import jax.numpy as jnp
from jax import lax
from jax.experimental import pallas as pl
from jax.experimental.pallas import tpu as pltpu

N_DEV = 4
N_TILES = 2
N_Q = 8
N_YSLOTS = 4

OWN, FROM_L, FROM_R, DIAG = 0, 1, 2, 3


def kernel(x, w_mat, scale_x, scale_w):
    m_per, k = x.shape
    q = m_per // N_Q
    n_total = w_mat.shape[1]
    n_per = n_total // N_DEV
    n_tile = n_per // N_TILES

    s = (scale_x[0] * scale_w[0]).reshape(1, 1)

    def body(x_hbm, w_hbm, s_ref, out_ref,
             comm_ref, w_ref, y_ref, send_sems, recv_sems, copy_sems,
             in_sems):
        my_pos = lax.axis_index("i")
        left = (my_pos - 1) % N_DEV
        right = (my_pos + 1) % N_DEV

        x_dma = pltpu.make_async_copy(x_hbm, comm_ref.at[OWN],
                                      in_sems.at[0])
        x_dma.start()
        w_dma = pltpu.make_async_copy(
            w_hbm.at[:, pl.ds(my_pos * n_per, n_per)], w_ref, in_sems.at[1])
        w_dma.start()

        barrier_sem = pltpu.get_barrier_semaphore()
        for nbr in [left, right]:
            pl.semaphore_signal(
                barrier_sem, inc=1,
                device_id=(nbr,), device_id_type=pl.DeviceIdType.MESH,
            )
        pl.semaphore_wait(barrier_sem, 2)

        scale = s_ref[0, 0]
        copies = [None] * (N_DEV * N_Q)
        unit = [0]

        def compute_q(origin, slot_idx, qi):
            c = unit[0]
            unit[0] += 1
            yslot = c % N_YSLOTS
            if c >= N_YSLOTS:
                copies[c - N_YSLOTS].wait()
            chunk = comm_ref[slot_idx, qi * q:(qi + 1) * q, :]
            for j in range(N_TILES):
                acc = lax.dot_general(
                    chunk, w_ref[:, j * n_tile:(j + 1) * n_tile],
                    (((1,), (0,)), ((), ())),
                    preferred_element_type=jnp.int32,
                )
                y_ref[yslot, :, j * n_tile:(j + 1) * n_tile] = jnp.maximum(
                    acc.astype(jnp.float32) * scale, 0.0)
            cp = pltpu.make_async_copy(
                y_ref.at[yslot],
                out_ref.at[pl.ds(origin * m_per + qi * q, q), :],
                copy_sems.at[c],
            )
            cp.start()
            copies[c] = cp

        def xfer(src_slot, dst_slot, qi, sem_idx, target):
            return pltpu.make_async_remote_copy(
                src_ref=comm_ref.at[src_slot, pl.ds(qi * q, q)],
                dst_ref=comm_ref.at[dst_slot, pl.ds(qi * q, q)],
                send_sem=send_sems.at[sem_idx],
                recv_sem=recv_sems.at[sem_idx],
                device_id=(target,),
                device_id_type=pl.DeviceIdType.MESH,
            )

        x_dma.wait()

        send_r = [xfer(OWN, FROM_L, qi, qi, right) for qi in range(N_Q)]
        send_l = [xfer(OWN, FROM_R, qi, N_Q + qi, left) for qi in range(N_Q)]
        for qi in range(N_Q):
            send_r[qi].start()
            send_l[N_Q - 1 - qi].start()

        w_dma.wait()
        for qi in range(N_Q):
            compute_q(my_pos, OWN, qi)

        relay = {}

        def on_left(qi):
            send_r[qi].wait_recv()
            if qi < N_Q // 2:
                relay[qi] = xfer(FROM_L, DIAG, qi, 2 * N_Q + qi, right)
                relay[qi].start()
            compute_q(left, FROM_L, qi)

        def on_right(qi):
            send_l[qi].wait_recv()
            if qi >= N_Q // 2:
                relay[qi] = xfer(FROM_R, DIAG, qi, 2 * N_Q + qi, left)
                relay[qi].start()
            compute_q(right, FROM_R, qi)

        for i in range(N_Q):
            on_left(i)
            on_right(N_Q - 1 - i)

        diag = (my_pos + 2) % N_DEV
        for i in range(N_Q // 2):
            for qi in (i, N_Q - 1 - i):
                relay[qi].wait_recv()
                compute_q(diag, DIAG, qi)

        for rdma in send_r + send_l + [relay[qi] for qi in range(N_Q)]:
            rdma.wait_send()
        for c in range(N_DEV * N_Q - N_YSLOTS, N_DEV * N_Q):
            copies[c].wait()

    return pl.pallas_call(
        body,
        out_shape=jax.ShapeDtypeStruct((N_DEV * m_per, n_per), jnp.float32),
        in_specs=[
            pl.BlockSpec(memory_space=pl.ANY),
            pl.BlockSpec(memory_space=pl.ANY),
            pl.BlockSpec(memory_space=pltpu.SMEM),
        ],
        out_specs=pl.BlockSpec(memory_space=pl.ANY),
        scratch_shapes=[
            pltpu.VMEM((N_DEV, m_per, k), x.dtype),
            pltpu.VMEM((k, n_per), w_mat.dtype),
            pltpu.VMEM((N_YSLOTS, q, n_per), jnp.float32),
            pltpu.SemaphoreType.DMA((3 * N_Q,)),
            pltpu.SemaphoreType.DMA((3 * N_Q,)),
            pltpu.SemaphoreType.DMA((N_DEV * N_Q,)),
            pltpu.SemaphoreType.DMA((2,)),
        ],
        compiler_params=pltpu.CompilerParams(
            collective_id=0,
            vmem_limit_bytes=60 * 1024 * 1024,
        ),
    )(x, w_mat, s)


# device time: 107052 ns/iter; 1.0306x vs baseline; 1.0306x over previous
import jax
import jax.numpy as jnp
from jax import lax
from jax.experimental import pallas as pl
from jax.experimental.pallas import tpu as pltpu

N_DEV = 4
N_TILES = 2
N_Q = 4
N_YSLOTS = 4

OWN, FROM_L, FROM_R, DIAG = 0, 1, 2, 3


def kernel(x, w_mat, scale_x, scale_w):
    m_per, k = x.shape
    q = m_per // N_Q
    n_total = w_mat.shape[1]
    n_per = n_total // N_DEV
    n_tile = n_per // N_TILES

    s = (scale_x[0] * scale_w[0]).reshape(1, 1)

    def body(x_hbm, w_hbm, s_ref, out_ref,
             comm_ref, w_ref, y_ref, send_sems, recv_sems, copy_sems,
             in_sems):
        my_pos = lax.axis_index("i")
        left = (my_pos - 1) % N_DEV
        right = (my_pos + 1) % N_DEV

        x_dma = pltpu.make_async_copy(x_hbm, comm_ref.at[OWN],
                                      in_sems.at[0])
        x_dma.start()
        w_dma = pltpu.make_async_copy(
            w_hbm.at[:, pl.ds(my_pos * n_per, n_per)], w_ref, in_sems.at[1])
        w_dma.start()

        barrier_sem = pltpu.get_barrier_semaphore()
        for nbr in [left, right]:
            pl.semaphore_signal(
                barrier_sem, inc=1,
                device_id=(nbr,), device_id_type=pl.DeviceIdType.MESH,
            )
        pl.semaphore_wait(barrier_sem, 2)

        scale = s_ref[0, 0]
        copies = [None] * (N_DEV * N_Q)
        unit = [0]

        def compute_q(origin, slot_idx, qi):
            c = unit[0]
            unit[0] += 1
            yslot = c % N_YSLOTS
            if c >= N_YSLOTS:
                copies[c - N_YSLOTS].wait()
            chunk = comm_ref[slot_idx, qi * q:(qi + 1) * q, :]
            for j in range(N_TILES):
                acc = lax.dot_general(
                    chunk, w_ref[:, j * n_tile:(j + 1) * n_tile],
                    (((1,), (0,)), ((), ())),
                    preferred_element_type=jnp.int32,
                )
                y_ref[yslot, :, j * n_tile:(j + 1) * n_tile] = jnp.maximum(
                    acc.astype(jnp.float32) * scale, 0.0)
            cp = pltpu.make_async_copy(
                y_ref.at[yslot],
                out_ref.at[pl.ds(origin * m_per + qi * q, q), :],
                copy_sems.at[c],
            )
            cp.start()
            copies[c] = cp

        def xfer(src_slot, dst_slot, qi, sem_idx, target):
            return pltpu.make_async_remote_copy(
                src_ref=comm_ref.at[src_slot, pl.ds(qi * q, q)],
                dst_ref=comm_ref.at[dst_slot, pl.ds(qi * q, q)],
                send_sem=send_sems.at[sem_idx],
                recv_sem=recv_sems.at[sem_idx],
                device_id=(target,),
                device_id_type=pl.DeviceIdType.MESH,
            )

        x_dma.wait()

        send_r = [xfer(OWN, FROM_L, qi, qi, right) for qi in range(N_Q)]
        send_l = [xfer(OWN, FROM_R, qi, N_Q + qi, left) for qi in range(N_Q)]
        for qi in range(N_Q):
            send_r[qi].start()
            send_l[N_Q - 1 - qi].start()

        w_dma.wait()
        for qi in range(N_Q):
            compute_q(my_pos, OWN, qi)

        relay = {}

        def on_left(qi):
            send_r[qi].wait_recv()
            if qi < N_Q // 2:
                relay[qi] = xfer(FROM_L, DIAG, qi, 2 * N_Q + qi, right)
                relay[qi].start()
            compute_q(left, FROM_L, qi)

        def on_right(qi):
            send_l[qi].wait_recv()
            if qi >= N_Q // 2:
                relay[qi] = xfer(FROM_R, DIAG, qi, 2 * N_Q + qi, left)
                relay[qi].start()
            compute_q(right, FROM_R, qi)

        for i in range(N_Q):
            on_left(i)
            on_right(N_Q - 1 - i)

        diag = (my_pos + 2) % N_DEV
        for i in range(N_Q // 2):
            for qi in (i, N_Q - 1 - i):
                relay[qi].wait_recv()
                compute_q(diag, DIAG, qi)

        for rdma in send_r + send_l + [relay[qi] for qi in range(N_Q)]:
            rdma.wait_send()
        for c in range(N_DEV * N_Q - N_YSLOTS, N_DEV * N_Q):
            copies[c].wait()

    return pl.pallas_call(
        body,
        out_shape=jax.ShapeDtypeStruct((N_DEV * m_per, n_per), jnp.float32),
        in_specs=[
            pl.BlockSpec(memory_space=pl.ANY),
            pl.BlockSpec(memory_space=pl.ANY),
            pl.BlockSpec(memory_space=pltpu.SMEM),
        ],
        out_specs=pl.BlockSpec(memory_space=pl.ANY),
        scratch_shapes=[
            pltpu.VMEM((N_DEV, m_per, k), x.dtype),
            pltpu.VMEM((k, n_per), w_mat.dtype),
            pltpu.VMEM((N_YSLOTS, q, n_per), jnp.float32),
            pltpu.SemaphoreType.DMA((3 * N_Q,)),
            pltpu.SemaphoreType.DMA((3 * N_Q,)),
            pltpu.SemaphoreType.DMA((N_DEV * N_Q,)),
            pltpu.SemaphoreType.DMA((2,)),
        ],
        compiler_params=pltpu.CompilerParams(
            collective_id=0,
            vmem_limit_bytes=60 * 1024 * 1024,
        ),
    )(x, w_mat, s)
